# baseline (speedup 1.0000x reference)
import jax
import jax.numpy as jnp
from jax.experimental import pallas as pl

_B, _S = 2, 8192
_HALF, _ROT = 64, 128
_THETA = 1000000.0
_N = _B * _S
_BLK = 1024
_H = _BLK // 2


def _dup(v):
    return jnp.concatenate(
        [v[:, :_HALF], v[:, :_HALF], v[:, _HALF:], v[:, _HALF:]], axis=-1)


def _rope_body(pos_ref, cos_ref, sin_ref):
    p2 = pos_ref[...].astype(jnp.float32)
    j = jax.lax.broadcasted_iota(jnp.int32, (1, _ROT), 1)
    j64 = (j & (_HALF - 1)).astype(jnp.float32)
    inv_freq2 = 1.0 / (_THETA ** (2.0 * j64 / _ROT))
    p0 = jnp.broadcast_to(p2[:, 0:1], (_H, _HALF))
    p1 = jnp.broadcast_to(p2[:, 1:2], (_H, _HALF))
    pbig = jnp.concatenate([p0, p1], axis=-1)
    f = pbig * inv_freq2
    cos_ref[...] = _dup(jnp.cos(f))
    sin_ref[...] = _dup(jnp.sin(f))


def kernel(x, position_ids):
    pos = position_ids.reshape(_N // 2, 2)
    cos, sin = pl.pallas_call(
        _rope_body,
        grid=(_N // _BLK,),
        in_specs=[pl.BlockSpec((_H, 2), lambda i: (i, 0))],
        out_specs=[pl.BlockSpec((_H, 2 * _ROT), lambda i: (i, 0))] * 2,
        out_shape=[jax.ShapeDtypeStruct((_N // 2, 2 * _ROT), jnp.float32)] * 2,
    )(pos)
    dt = x.dtype
    return (cos.reshape(_B, _S, _ROT).astype(dt), sin.reshape(_B, _S, _ROT).astype(dt))


# speedup vs baseline: 1.8929x; 1.8929x over previous
import jax
import jax.numpy as jnp
from jax.experimental import pallas as pl

_B, _S = 2, 8192
_HALF, _ROT = 64, 128
_THETA = 1000000.0
_N = _B * _S
_GRID = 16
_ROWS = 8
_BLK = _ROWS * 128


def _rope_body(pos_ref, cos_ref, sin_ref):
    jcol = jax.lax.broadcasted_iota(jnp.int32, (_HALF, 1), 0).astype(jnp.float32)
    inv_freq_col = 1.0 / (_THETA ** (2.0 * jcol / _ROT))
    for r in range(_ROWS):
        p = pos_ref[0, r, :].astype(jnp.float32)
        pt = jnp.broadcast_to(p.reshape(1, 128), (_HALF, 128))
        ft = pt * inv_freq_col
        ct = jnp.cos(ft).T
        st = jnp.sin(ft).T
        cos_ref[pl.ds(r * 128, 128), :] = jnp.concatenate([ct, ct], axis=-1)
        sin_ref[pl.ds(r * 128, 128), :] = jnp.concatenate([st, st], axis=-1)


def kernel(x, position_ids):
    pos = position_ids.reshape(_GRID, _ROWS, 128)
    cos, sin = pl.pallas_call(
        _rope_body,
        grid=(_GRID,),
        in_specs=[pl.BlockSpec((1, _ROWS, 128), lambda i: (i, 0, 0))],
        out_specs=[pl.BlockSpec((_BLK, _ROT), lambda i: (i, 0))] * 2,
        out_shape=[jax.ShapeDtypeStruct((_N, _ROT), jnp.float32)] * 2,
    )(pos)
    dt = x.dtype
    return (cos.reshape(_B, _S, _ROT).astype(dt), sin.reshape(_B, _S, _ROT).astype(dt))


# speedup vs baseline: 1.9581x; 1.0344x over previous
import jax
import jax.numpy as jnp
from jax.experimental import pallas as pl

_B, _S = 2, 8192
_HALF, _ROT = 64, 128
_THETA = 1000000.0
_N = _B * _S
_GRID = 8
_ROWS = 16
_BLK = _ROWS * 128


def _rope_body(pos_ref, cos_ref, sin_ref):
    jcol = jax.lax.broadcasted_iota(jnp.int32, (_HALF, 1), 0).astype(jnp.float32)
    inv_freq_col = 1.0 / (_THETA ** (2.0 * jcol / _ROT))
    for r in range(_ROWS):
        p = pos_ref[0, r, :].astype(jnp.float32)
        pt = jnp.broadcast_to(p.reshape(1, 128), (_HALF, 128))
        ft = pt * inv_freq_col
        ct = jnp.cos(ft).T
        st = jnp.sin(ft).T
        cos_ref[pl.ds(r * 128, 128), :] = jnp.concatenate([ct, ct], axis=-1)
        sin_ref[pl.ds(r * 128, 128), :] = jnp.concatenate([st, st], axis=-1)


def kernel(x, position_ids):
    pos = position_ids.reshape(_GRID, _ROWS, 128)
    cos, sin = pl.pallas_call(
        _rope_body,
        grid=(_GRID,),
        in_specs=[pl.BlockSpec((1, _ROWS, 128), lambda i: (i, 0, 0))],
        out_specs=[pl.BlockSpec((_BLK, _ROT), lambda i: (i, 0))] * 2,
        out_shape=[jax.ShapeDtypeStruct((_N, _ROT), jnp.float32)] * 2,
    )(pos)
    dt = x.dtype
    return (cos.reshape(_B, _S, _ROT).astype(dt), sin.reshape(_B, _S, _ROT).astype(dt))
